# baseline (speedup 1.0000x reference)
import functools

import jax
import jax.numpy as jnp
from jax import lax
from jax.experimental import pallas as pl
from jax.experimental.pallas import tpu as pltpu
from jax.experimental.pallas import tpu_sc as plsc

B = 16384
K = 64
D = 128
L = 16

NC = 2
NS = 16
NW = NC * NS
BPW = B // NW
G = 128
NG = BPW // G
CB = 1
NCHUNK = G // CB
NBUF = 8
KG = K // L
KUNROLL = 16


def _sc_body(pos_u, pos_v, neg2d, uw, vw, pos_out, neg_out,
             uidx, vidx, nidx_all, urows, vrows, *rest):
    nbufs = rest[0:NBUF]
    nstages = rest[NBUF:2 * NBUF]
    pos_stage = rest[2 * NBUF]
    sem_u = rest[2 * NBUF + 1]
    sem_v = rest[2 * NBUF + 2]
    gsems = rest[2 * NBUF + 3:2 * NBUF + 3 + NBUF]
    wsems = rest[2 * NBUF + 3 + NBUF:2 * NBUF + 3 + 2 * NBUF]

    cid = lax.axis_index("c")
    sid = lax.axis_index("s")
    wid = sid * NC + cid
    base = wid * BPW

    iota = lax.iota(jnp.int32, L)
    mask_last = iota == (L - 1)
    krows = [iota + (kg * L) for kg in range(KG)]

    def compute_chunk(c, buf, stage):
        uvec = [urows[c, pl.ds(j * L, L)] for j in range(D // L)]
        p0 = uvec[0] * vrows[c, pl.ds(0, L)]
        p1 = uvec[1] * vrows[c, pl.ds(L, L)]
        for j in range(2, D // L, 2):
            p0 = p0 + uvec[j] * vrows[c, pl.ds(j * L, L)]
            p1 = p1 + uvec[j + 1] * vrows[c, pl.ds((j + 1) * L, L)]
        plsc.store_scatter(pos_stage, [jnp.full((L,), c, jnp.int32)],
                           plsc.cumsum(p0 + p1), mask=mask_last)

        def row_dot(row):
            q0 = uvec[0] * buf[row, pl.ds(0, L)]
            q1 = uvec[1] * buf[row, pl.ds(L, L)]
            for j in range(2, D // L, 2):
                q0 = q0 + uvec[j] * buf[row, pl.ds(j * L, L)]
                q1 = q1 + uvec[j + 1] * buf[row, pl.ds((j + 1) * L, L)]
            return q0 + q1

        def kbody(kq, tgt):
            for kk in range(0, KUNROLL, 16):
                r = kq * KUNROLL + kk
                ds_ = [row_dot(r + i) for i in range(16)]
                for i in range(16):
                    plsc.store_scatter(stage, [tgt + i],
                                       plsc.cumsum(ds_[i]), mask=mask_last)
                tgt = tgt + 16
            return tgt

        lax.fori_loop(0, K // KUNROLL, kbody, jnp.zeros((L,), jnp.int32))

    def group_body(g, _):
        gbase = base + g * G
        pltpu.sync_copy(pos_u.at[pl.ds(gbase, G)], uidx)
        pltpu.sync_copy(pos_v.at[pl.ds(gbase, G)], vidx)
        cu = pltpu.async_copy(uw.at[uidx], urows, sem_u)
        cv = pltpu.async_copy(vw.at[vidx], vrows, sem_v)
        row0 = pl.multiple_of(gbase // CB, NCHUNK)
        pltpu.sync_copy(neg2d.at[pl.ds(row0, NCHUNK)], nidx_all)
        cu.wait()
        cv.wait()

        for j in range(NBUF):
            pltpu.async_copy(vw.at[nidx_all.at[j]], nbufs[j], gsems[j])

        def outer(co, _):
            for j in range(NBUF):
                c = co * NBUF + j
                pltpu.make_async_copy(
                    vw.at[nidx_all.at[c]], nbufs[j], gsems[j]).wait()

                @pl.when(co > 0)
                def _wait_wb():
                    pltpu.make_async_copy(
                        nstages[j], neg_out.at[pl.ds(0, CB * K)],
                        wsems[j]).wait()

                compute_chunk(c, nbufs[j], nstages[j])
                b0 = gbase + c * CB
                pltpu.async_copy(
                    nstages[j], neg_out.at[pl.ds(b0 * K, CB * K)], wsems[j])

                @pl.when(c + NBUF < NCHUNK)
                def _refill():
                    pltpu.async_copy(
                        vw.at[nidx_all.at[c + NBUF]], nbufs[j], gsems[j])
            return 0

        lax.fori_loop(0, NCHUNK // NBUF, outer, 0)
        for j in range(NBUF):
            pltpu.make_async_copy(
                nstages[j], neg_out.at[pl.ds(0, CB * K)], wsems[j]).wait()
        pltpu.sync_copy(pos_stage, pos_out.at[pl.ds(gbase, G)])
        return 0

    lax.fori_loop(0, NG, group_body, 0)


_sc_dots = functools.partial(
    pl.kernel,
    out_type=[
        jax.ShapeDtypeStruct((B,), jnp.float32),
        jax.ShapeDtypeStruct((B * K,), jnp.float32),
    ],
    mesh=plsc.VectorSubcoreMesh(core_axis_name="c", subcore_axis_name="s"),
    compiler_params=pltpu.CompilerParams(needs_layout_passes=False,
                                         use_tc_tiling_on_sc=False),
    scratch_types=[
        pltpu.VMEM((G,), jnp.int32),
        pltpu.VMEM((G,), jnp.int32),
        pltpu.VMEM((NCHUNK, CB * K), jnp.int32),
        pltpu.VMEM((G, D), jnp.float32),
        pltpu.VMEM((G, D), jnp.float32),
    ]
    + [pltpu.VMEM((CB * K, D), jnp.float32) for _ in range(NBUF)]
    + [pltpu.VMEM((CB * K,), jnp.float32) for _ in range(NBUF)]
    + [
        pltpu.VMEM((G,), jnp.float32),
        pltpu.SemaphoreType.DMA,
        pltpu.SemaphoreType.DMA,
    ]
    + [pltpu.SemaphoreType.DMA for _ in range(2 * NBUF)],
)(_sc_body)


def _loss_body(pos_ref, neg_ref, out_ref):
    s = jnp.sum(jax.nn.log_sigmoid(pos_ref[...]))
    s = s + jnp.sum(jax.nn.log_sigmoid(-neg_ref[...]))
    out_ref[...] = jnp.full((1, 1), -s, jnp.float32)


def kernel(pos_u, pos_v, neg_v, u_weight, v_weight):
    pos_u = pos_u.astype(jnp.int32)
    pos_v = pos_v.astype(jnp.int32)
    neg2d = neg_v.astype(jnp.int32).reshape(B * K // (CB * K), CB * K)
    pos_dot, neg_dot = _sc_dots(pos_u, pos_v, neg2d, u_weight, v_weight)
    loss2d = pl.pallas_call(
        _loss_body,
        out_shape=jax.ShapeDtypeStruct((1, 1), jnp.float32),
    )(pos_dot.reshape(B // D, D), neg_dot.reshape(B * K // D, D))
    return loss2d[0, 0]


# speedup vs baseline: 1.1431x; 1.1431x over previous
import functools

import jax
import jax.numpy as jnp
from jax import lax
from jax.experimental import pallas as pl
from jax.experimental.pallas import tpu as pltpu
from jax.experimental.pallas import tpu_sc as plsc

B = 16384
K = 64
D = 128
L = 16

NC = 2
NS = 16
NW = NC * NS
BPW = B // NW
G = 256
NG = BPW // G
CB = 1
NCHUNK = G // CB
NBUF = 4
KG = K // L
KUNROLL = 16


def _sc_body(pos_u, pos_v, neg2d, uw, vw, pos_out, neg_out,
             uidx, vidx, nidx_all, urows, vrows, *rest):
    nbufs = rest[0:NBUF]
    nstages = rest[NBUF:2 * NBUF]
    pos_stage = rest[2 * NBUF]
    sem_u = rest[2 * NBUF + 1]
    sem_v = rest[2 * NBUF + 2]
    gsems = rest[2 * NBUF + 3:2 * NBUF + 3 + NBUF]
    wsems = rest[2 * NBUF + 3 + NBUF:2 * NBUF + 3 + 2 * NBUF]

    cid = lax.axis_index("c")
    sid = lax.axis_index("s")
    wid = sid * NC + cid
    base = wid * BPW

    iota = lax.iota(jnp.int32, L)
    mask_last = iota == (L - 1)
    krows = [iota + (kg * L) for kg in range(KG)]

    def compute_chunk(c, buf, stage):
        uvec = [urows[c, pl.ds(j * L, L)] for j in range(D // L)]
        p0 = uvec[0] * vrows[c, pl.ds(0, L)]
        p1 = uvec[1] * vrows[c, pl.ds(L, L)]
        for j in range(2, D // L, 2):
            p0 = p0 + uvec[j] * vrows[c, pl.ds(j * L, L)]
            p1 = p1 + uvec[j + 1] * vrows[c, pl.ds((j + 1) * L, L)]
        plsc.store_scatter(pos_stage, [jnp.full((L,), c, jnp.int32)],
                           plsc.cumsum(p0 + p1), mask=mask_last)

        def row_dot(row):
            q0 = uvec[0] * buf[row, pl.ds(0, L)]
            q1 = uvec[1] * buf[row, pl.ds(L, L)]
            for j in range(2, D // L, 2):
                q0 = q0 + uvec[j] * buf[row, pl.ds(j * L, L)]
                q1 = q1 + uvec[j + 1] * buf[row, pl.ds((j + 1) * L, L)]
            return q0 + q1

        def kbody(kq, tgt):
            for kk in range(0, KUNROLL, 16):
                r = kq * KUNROLL + kk
                ds_ = [row_dot(r + i) for i in range(16)]
                for i in range(16):
                    plsc.store_scatter(stage, [tgt + i],
                                       plsc.cumsum(ds_[i]), mask=mask_last)
                tgt = tgt + 16
            return tgt

        lax.fori_loop(0, K // KUNROLL, kbody, jnp.zeros((L,), jnp.int32))

    def group_body(g, _):
        gbase = base + g * G
        pltpu.sync_copy(pos_u.at[pl.ds(gbase, G)], uidx)
        pltpu.sync_copy(pos_v.at[pl.ds(gbase, G)], vidx)
        cus = [pltpu.async_copy(uw.at[uidx.at[pl.ds(h * 128, 128)]],
                                urows.at[pl.ds(h * 128, 128)], sem_u)
               for h in range(G // 128)]
        cvs = [pltpu.async_copy(vw.at[vidx.at[pl.ds(h * 128, 128)]],
                                vrows.at[pl.ds(h * 128, 128)], sem_v)
               for h in range(G // 128)]
        row0 = pl.multiple_of(gbase // CB, NCHUNK)
        pltpu.sync_copy(neg2d.at[pl.ds(row0, NCHUNK)], nidx_all)
        for cp in cus + cvs:
            cp.wait()

        for j in range(NBUF):
            pltpu.async_copy(vw.at[nidx_all.at[j]], nbufs[j], gsems[j])

        def outer(co, _):
            for j in range(NBUF):
                c = co * NBUF + j
                pltpu.make_async_copy(
                    vw.at[nidx_all.at[c]], nbufs[j], gsems[j]).wait()

                @pl.when(co > 0)
                def _wait_wb():
                    pltpu.make_async_copy(
                        nstages[j], neg_out.at[pl.ds(0, CB * K)],
                        wsems[j]).wait()

                compute_chunk(c, nbufs[j], nstages[j])
                b0 = gbase + c * CB
                pltpu.async_copy(
                    nstages[j], neg_out.at[pl.ds(b0 * K, CB * K)], wsems[j])

                @pl.when(c + NBUF < NCHUNK)
                def _refill():
                    pltpu.async_copy(
                        vw.at[nidx_all.at[c + NBUF]], nbufs[j], gsems[j])
            return 0

        lax.fori_loop(0, NCHUNK // NBUF, outer, 0)
        for j in range(NBUF):
            pltpu.make_async_copy(
                nstages[j], neg_out.at[pl.ds(0, CB * K)], wsems[j]).wait()
        pltpu.sync_copy(pos_stage, pos_out.at[pl.ds(gbase, G)])
        return 0

    lax.fori_loop(0, NG, group_body, 0)


_sc_dots = functools.partial(
    pl.kernel,
    out_type=[
        jax.ShapeDtypeStruct((B,), jnp.float32),
        jax.ShapeDtypeStruct((B * K,), jnp.float32),
    ],
    mesh=plsc.VectorSubcoreMesh(core_axis_name="c", subcore_axis_name="s"),
    compiler_params=pltpu.CompilerParams(needs_layout_passes=False,
                                         use_tc_tiling_on_sc=False),
    scratch_types=[
        pltpu.VMEM((G,), jnp.int32),
        pltpu.VMEM((G,), jnp.int32),
        pltpu.VMEM((NCHUNK, CB * K), jnp.int32),
        pltpu.VMEM((G, D), jnp.float32),
        pltpu.VMEM((G, D), jnp.float32),
    ]
    + [pltpu.VMEM((CB * K, D), jnp.float32) for _ in range(NBUF)]
    + [pltpu.VMEM((CB * K,), jnp.float32) for _ in range(NBUF)]
    + [
        pltpu.VMEM((G,), jnp.float32),
        pltpu.SemaphoreType.DMA,
        pltpu.SemaphoreType.DMA,
    ]
    + [pltpu.SemaphoreType.DMA for _ in range(2 * NBUF)],
)(_sc_body)


def _loss_body(pos_ref, neg_ref, out_ref):
    s = jnp.sum(jax.nn.log_sigmoid(pos_ref[...]))
    s = s + jnp.sum(jax.nn.log_sigmoid(-neg_ref[...]))
    out_ref[...] = jnp.full((1, 1), -s, jnp.float32)


def kernel(pos_u, pos_v, neg_v, u_weight, v_weight):
    pos_u = pos_u.astype(jnp.int32)
    pos_v = pos_v.astype(jnp.int32)
    neg2d = neg_v.astype(jnp.int32).reshape(B * K // (CB * K), CB * K)
    pos_dot, neg_dot = _sc_dots(pos_u, pos_v, neg2d, u_weight, v_weight)
    loss2d = pl.pallas_call(
        _loss_body,
        out_shape=jax.ShapeDtypeStruct((1, 1), jnp.float32),
    )(pos_dot.reshape(B // D, D), neg_dot.reshape(B * K // D, D))
    return loss2d[0, 0]


# speedup vs baseline: 1.1459x; 1.0024x over previous
import functools

import jax
import jax.numpy as jnp
from jax import lax
from jax.experimental import pallas as pl
from jax.experimental.pallas import tpu as pltpu
from jax.experimental.pallas import tpu_sc as plsc

B = 16384
K = 64
D = 128
L = 16

NC = 2
NS = 16
NW = NC * NS
BPW = B // NW
G = 256
NG = BPW // G
CB = 1
NCHUNK = G // CB
NBUF = 4
KUNROLL = 16


def _sc_body(pos_u, pos_v, neg2d, uw, vw, pos_out, neg_out,
             uidx, vidx, nidx_all, urows, vrows, *rest):
    nbufs = rest[0:NBUF]
    nstages = rest[NBUF:2 * NBUF]
    pos_stage = rest[2 * NBUF]
    sem_u = rest[2 * NBUF + 1]
    sem_v = rest[2 * NBUF + 2]
    gsems = rest[2 * NBUF + 3:2 * NBUF + 3 + NBUF]
    wsems = rest[2 * NBUF + 3 + NBUF:2 * NBUF + 3 + 2 * NBUF]

    cid = lax.axis_index("c")
    sid = lax.axis_index("s")
    wid = sid * NC + cid
    base = wid * BPW

    iota = lax.iota(jnp.int32, L)
    mask_last = iota == (L - 1)

    def compute_chunk(c, buf, stage):
        uvec = [urows[c, pl.ds(j * L, L)] for j in range(D // L)]
        p0 = uvec[0] * vrows[c, pl.ds(0, L)]
        p1 = uvec[1] * vrows[c, pl.ds(L, L)]
        for j in range(2, D // L, 2):
            p0 = p0 + uvec[j] * vrows[c, pl.ds(j * L, L)]
            p1 = p1 + uvec[j + 1] * vrows[c, pl.ds((j + 1) * L, L)]
        plsc.store_scatter(pos_stage, [jnp.full((L,), c, jnp.int32)],
                           plsc.cumsum(p0 + p1), mask=mask_last)

        def row_dot(row):
            q0 = uvec[0] * buf[row, pl.ds(0, L)]
            q1 = uvec[1] * buf[row, pl.ds(L, L)]
            for j in range(2, D // L, 2):
                q0 = q0 + uvec[j] * buf[row, pl.ds(j * L, L)]
                q1 = q1 + uvec[j + 1] * buf[row, pl.ds((j + 1) * L, L)]
            return q0 + q1

        def kbody(kq, tgt):
            for kk in range(0, KUNROLL, 16):
                r = kq * KUNROLL + kk
                ds_ = [row_dot(r + i) for i in range(16)]
                for i in range(16):
                    plsc.store_scatter(stage, [tgt + i],
                                       plsc.cumsum(ds_[i]), mask=mask_last)
                tgt = tgt + 16
            return tgt

        lax.fori_loop(0, K // KUNROLL, kbody, jnp.zeros((L,), jnp.int32))

    def group_body(g, _):
        gbase = base + g * G
        pltpu.sync_copy(pos_u.at[pl.ds(gbase, G)], uidx)
        pltpu.sync_copy(pos_v.at[pl.ds(gbase, G)], vidx)
        cus = [pltpu.async_copy(uw.at[uidx.at[pl.ds(h * 128, 128)]],
                                urows.at[pl.ds(h * 128, 128)], sem_u)
               for h in range(G // 128)]
        cvs = [pltpu.async_copy(vw.at[vidx.at[pl.ds(h * 128, 128)]],
                                vrows.at[pl.ds(h * 128, 128)], sem_v)
               for h in range(G // 128)]
        row0 = pl.multiple_of(gbase // CB, NCHUNK)
        pltpu.sync_copy(neg2d.at[pl.ds(row0, NCHUNK)], nidx_all)
        for cp in cus + cvs:
            cp.wait()

        for j in range(NBUF):
            pltpu.async_copy(vw.at[nidx_all.at[j]], nbufs[j], gsems[j])

        def outer(co, _):
            for j in range(NBUF):
                c = co * NBUF + j
                pltpu.make_async_copy(
                    vw.at[nidx_all.at[c]], nbufs[j], gsems[j]).wait()

                @pl.when(co > 0)
                def _wait_wb():
                    pltpu.make_async_copy(
                        nstages[j], neg_out.at[pl.ds(0, CB * K)],
                        wsems[j]).wait()

                compute_chunk(c, nbufs[j], nstages[j])
                b0 = gbase + c * CB
                pltpu.async_copy(
                    nstages[j], neg_out.at[pl.ds(b0 * K, CB * K)], wsems[j])

                @pl.when(c + NBUF < NCHUNK)
                def _refill():
                    pltpu.async_copy(
                        vw.at[nidx_all.at[c + NBUF]], nbufs[j], gsems[j])
            return 0

        lax.fori_loop(0, NCHUNK // NBUF, outer, 0)
        for j in range(NBUF):
            pltpu.make_async_copy(
                nstages[j], neg_out.at[pl.ds(0, CB * K)], wsems[j]).wait()
        pltpu.sync_copy(pos_stage, pos_out.at[pl.ds(gbase, G)])
        return 0

    lax.fori_loop(0, NG, group_body, 0)


_sc_dots = functools.partial(
    pl.kernel,
    out_type=[
        jax.ShapeDtypeStruct((B,), jnp.float32),
        jax.ShapeDtypeStruct((B * K,), jnp.float32),
    ],
    mesh=plsc.VectorSubcoreMesh(core_axis_name="c", subcore_axis_name="s"),
    compiler_params=pltpu.CompilerParams(needs_layout_passes=False,
                                         use_tc_tiling_on_sc=False),
    scratch_types=[
        pltpu.VMEM((G,), jnp.int32),
        pltpu.VMEM((G,), jnp.int32),
        pltpu.VMEM((NCHUNK, CB * K), jnp.int32),
        pltpu.VMEM((G, D), jnp.float32),
        pltpu.VMEM((G, D), jnp.float32),
    ]
    + [pltpu.VMEM((CB * K, D), jnp.float32) for _ in range(NBUF)]
    + [pltpu.VMEM((CB * K,), jnp.float32) for _ in range(NBUF)]
    + [
        pltpu.VMEM((G,), jnp.float32),
        pltpu.SemaphoreType.DMA,
        pltpu.SemaphoreType.DMA,
    ]
    + [pltpu.SemaphoreType.DMA for _ in range(2 * NBUF)],
)(_sc_body)


def _loss_body(pos_ref, neg_ref, out_ref):
    s = jnp.sum(jax.nn.log_sigmoid(pos_ref[...]))
    s = s + jnp.sum(jax.nn.log_sigmoid(-neg_ref[...]))
    out_ref[...] = jnp.full((1, 1), -s, jnp.float32)


def kernel(pos_u, pos_v, neg_v, u_weight, v_weight):
    pos_u = pos_u.astype(jnp.int32)
    pos_v = pos_v.astype(jnp.int32)
    neg2d = neg_v.astype(jnp.int32).reshape(B * K // (CB * K), CB * K)
    pos_dot, neg_dot = _sc_dots(pos_u, pos_v, neg2d, u_weight, v_weight)
    loss2d = pl.pallas_call(
        _loss_body,
        out_shape=jax.ShapeDtypeStruct((1, 1), jnp.float32),
    )(pos_dot.reshape(B // D, D), neg_dot.reshape(B * K // D, D))
    return loss2d[0, 0]
